# Initial kernel scaffold; baseline (speedup 1.0000x reference)
import jax
import jax.numpy as jnp
from jax.experimental import pallas as pl


def kernel(population, target_prob):
    raise NotImplementedError("write your pallas kernel here")


# speedup vs baseline: 1.0031x; 1.0031x over previous
import functools
import math

import jax
import jax.numpy as jnp
from jax.experimental import pallas as pl
from jax.experimental.pallas import tpu as pltpu


def _roll_lanes(x, shift):
    return jnp.concatenate([x[:, shift:], x[:, :shift]], axis=1)


def _penalty_body(xs_ref, ys_ref, out_ref, *, grid_size):
    xs = xs_ref[...]
    ys = ys_ref[...]
    blk, n = xs.shape

    dx = jnp.abs(xs[:, 1:] - xs[:, :-1])
    dy = jnp.abs(ys[:, 1:] - ys[:, :-1])
    cont = jnp.sum((dx + dy > 1.0).astype(jnp.float32), axis=1)

    x = xs.astype(jnp.int32) * grid_size + ys.astype(jnp.int32)

    log_n = int(math.log2(n))
    assert (1 << log_n) == n
    idx = jax.lax.broadcasted_iota(jnp.int32, (1, n), 1)
    for k in range(1, log_n + 1):
        for j in range(k - 1, -1, -1):
            d = 1 << j
            fwd = _roll_lanes(x, d)
            bwd = _roll_lanes(x, n - d)
            is_lo = (idx & d) == 0
            partner = jnp.where(is_lo, fwd, bwd)
            asc = (idx & (1 << k)) == 0
            take_min = asc == is_lo
            x = jnp.where(take_min, jnp.minimum(x, partner),
                          jnp.maximum(x, partner))

    adj = x[:, 1:] == x[:, :-1]
    rep = (jnp.sum(adj.astype(jnp.float32), axis=1)
           - jnp.sum((adj[:, 1:] & adj[:, :-1]).astype(jnp.float32), axis=1))

    out_ref[...] = cont * 0.5 + rep * 0.2


def _penalties(xs, ys, grid_size, block_rows):
    pop, n = xs.shape
    grid = pop // block_rows
    return pl.pallas_call(
        functools.partial(_penalty_body, grid_size=grid_size),
        grid=(grid,),
        in_specs=[
            pl.BlockSpec((block_rows, n), lambda i: (i, 0)),
            pl.BlockSpec((block_rows, n), lambda i: (i, 0)),
        ],
        out_specs=pl.BlockSpec((block_rows,), lambda i: (i,)),
        out_shape=jax.ShapeDtypeStruct((pop,), jnp.float32),
    )(xs, ys)


def kernel(population, target_prob):
    pop, n, _ = population.shape
    grid_size = target_prob.shape[0]
    xs = population[:, :, 0]
    ys = population[:, :, 1]

    pen = _penalties(xs, ys, grid_size, block_rows=min(128, pop))

    pts = population.reshape(-1, 2).astype(jnp.int32)
    probs = target_prob[pts[:, 0], pts[:, 1]].reshape(pop, n)
    return probs.sum(axis=1) - pen
